# baseline (speedup 1.0000x reference)
import jax
import jax.numpy as jnp
import numpy as np
from jax import lax
from jax.experimental import pallas as pl
from jax.experimental.pallas import tpu as pltpu
from jax.experimental.pallas import tpu_sc as plsc

N_ROWS = 1048576
N_DIMS = 16
NC, NS = 2, 16
NW = NC * NS
ROWS_PER_W = N_ROWS // NW
CH = 512
NG = CH // N_DIMS
N_CHUNKS = ROWS_PER_W // CH
PAD = N_DIMS + 1
K_SUP = float(np.sqrt(-np.log(0.01)))
K2 = K_SUP * K_SUP
PENALTY = 200.0


def _sc_body(x_hbm, c_hbm, w_hbm, m_hbm, out_hbm,
             xb, yb, pb, si0, si1, so0, so1):
    wid = lax.axis_index("s") * NC + lax.axis_index("c")
    slab = wid * ROWS_PER_W

    pltpu.sync_copy(c_hbm, pb.at[0])
    pltpu.sync_copy(w_hbm, pb.at[1])
    pltpu.sync_copy(m_hbm, pb.at[2])
    cv = pb[0]
    wv = pb[1]
    iwv = 1.0 / wv
    k2v = jnp.where(wv > 0.0, jnp.float32(K2), jnp.float32(-1.0))
    cs = [cv[d] for d in range(N_DIMS)]
    iws = [iwv[d] for d in range(N_DIMS)]
    k2s = [k2v[d] for d in range(N_DIMS)]
    ms = pb[2][0]
    row_iota = lax.iota(jnp.int32, N_DIMS)
    cols = [row_iota * 0 + d for d in range(N_DIMS)]
    sems_i = (si0, si1)
    sems_o = (so0, so1)

    def make_group_body(par):
        parv = row_iota * 0 + par

        def group_body(j, _):
            base_v = row_iota * 0 + j * N_DIMS + row_iota
            qq = None
            for d in range(N_DIMS):
                v = plsc.load_gather(xb, [parv, base_v, cols[d]])
                t = (v - cs[d]) * iws[d]
                q = t * t
                qqd = jnp.where(q < k2s[d], q, jnp.float32(PENALTY))
                qq = qqd if qq is None else qq + qqd
            yb[par, pl.ds(j * N_DIMS, N_DIMS)] = ms * jnp.exp(-qq)
            return 0

        return group_body

    pltpu.async_copy(x_hbm.at[pl.ds(slab, CH)],
                     xb.at[0, :, pl.ds(0, N_DIMS)], si0)

    def pair_body(p, _):
        for par in range(2):
            k = 2 * p + par
            base = slab + k * CH
            @pl.when(k + 1 < N_CHUNKS)
            def _():
                pltpu.async_copy(
                    x_hbm.at[pl.ds(base + CH, CH)],
                    xb.at[1 - par, :, pl.ds(0, N_DIMS)],
                    sems_i[1 - par])
            pltpu.make_async_copy(
                x_hbm.at[pl.ds(base, CH)],
                xb.at[par, :, pl.ds(0, N_DIMS)], sems_i[par]).wait()
            @pl.when(k >= 2)
            def _():
                pltpu.make_async_copy(
                    yb.at[par], out_hbm.at[pl.ds(base, CH)],
                    sems_o[par]).wait()
            plsc.parallel_loop(0, NG, 1, unroll=2, carry=jnp.int32(0))(
                make_group_body(par))
            pltpu.async_copy(yb.at[par], out_hbm.at[pl.ds(base, CH)],
                             sems_o[par])
        return 0

    lax.fori_loop(0, N_CHUNKS // 2, pair_body, 0)
    for par in range(2):
        base = slab + (N_CHUNKS - 2 + par) * CH
        pltpu.make_async_copy(
            yb.at[par], out_hbm.at[pl.ds(base, CH)], sems_o[par]).wait()


def kernel(x, c, w, m):
    m16 = jnp.broadcast_to(m, (N_DIMS,))
    mesh = plsc.VectorSubcoreMesh(core_axis_name="c", subcore_axis_name="s")
    f = pl.kernel(
        _sc_body,
        out_type=jax.ShapeDtypeStruct((N_ROWS,), jnp.float32),
        mesh=mesh,
        compiler_params=pltpu.CompilerParams(
            needs_layout_passes=False, use_tc_tiling_on_sc=False),
        scratch_types=[
            pltpu.VMEM((2, CH, PAD), jnp.float32),
            pltpu.VMEM((2, CH), jnp.float32),
            pltpu.VMEM((3, N_DIMS), jnp.float32),
            pltpu.SemaphoreType.DMA,
            pltpu.SemaphoreType.DMA,
            pltpu.SemaphoreType.DMA,
            pltpu.SemaphoreType.DMA,
        ],
    )
    return f(x, c, w, m16)


# speedup vs baseline: 1.2861x; 1.2861x over previous
import jax
import jax.numpy as jnp
import numpy as np
from jax import lax
from jax.experimental import pallas as pl
from jax.experimental.pallas import tpu as pltpu
from jax.experimental.pallas import tpu_sc as plsc

N_ROWS = 1048576
N_DIMS = 16
NC, NS = 2, 16
NW = NC * NS
ROWS_PER_W = N_ROWS // NW
CH = 512
NG = CH // N_DIMS
N_CHUNKS = ROWS_PER_W // CH
K_SUP = float(np.sqrt(-np.log(0.01)))
K2 = K_SUP * K_SUP
PENALTY = 200.0


def _sc_body(x_hbm, c_hbm, w_hbm, m_hbm, out_hbm,
             xb, yb, pb, sb, si0, si1, so0, so1):
    wid = lax.axis_index("s") * NC + lax.axis_index("c")
    slab = wid * ROWS_PER_W

    pltpu.sync_copy(c_hbm, pb.at[0])
    pltpu.sync_copy(w_hbm, pb.at[1])
    pltpu.sync_copy(m_hbm, pb.at[2])
    cv = pb[0]
    wv = pb[1]
    mv = pb[2]
    iw = 1.0 / wv
    k2v = jnp.where(wv > 0.0, jnp.float32(K2), jnp.float32(-1.0))
    row_iota = lax.iota(jnp.int32, N_DIMS)
    cols = [row_iota * 0 + d for d in range(N_DIMS)]
    sems_i = (si0, si1)
    sems_o = (so0, so1)

    def make_group_body(par):
        parv = row_iota * 0 + par

        def group_body(j, _):
            jbase = j * N_DIMS
            vs = [xb[par, jbase + r] for r in range(N_DIMS)]
            ts = [(v - cv) * iw for v in vs]
            qs = [t * t for t in ts]
            qqs = [jnp.where(q < k2v, q, jnp.float32(PENALTY)) for q in qs]
            for r in range(N_DIMS):
                sb[par, j, r, pl.ds(0, N_DIMS)] = qqs[r]
            jv = row_iota * 0 + j
            g = [plsc.load_gather(sb, [parv, jv, row_iota, cols[d]])
                 for d in range(N_DIMS)]
            while len(g) > 1:
                g = [g[i] + g[i + 1] for i in range(0, len(g), 2)]
            yb[par, pl.ds(jbase, N_DIMS)] = mv * jnp.exp(-g[0])
            return 0

        return group_body

    pltpu.async_copy(x_hbm.at[pl.ds(slab, CH)], xb.at[0], si0)

    def pair_body(p, _):
        for par in range(2):
            k = 2 * p + par
            base = slab + k * CH
            @pl.when(k + 1 < N_CHUNKS)
            def _():
                pltpu.async_copy(
                    x_hbm.at[pl.ds(base + CH, CH)], xb.at[1 - par],
                    sems_i[1 - par])
            pltpu.make_async_copy(
                x_hbm.at[pl.ds(base, CH)], xb.at[par], sems_i[par]).wait()
            @pl.when(k >= 2)
            def _():
                pltpu.make_async_copy(
                    yb.at[par], out_hbm.at[pl.ds(base, CH)],
                    sems_o[par]).wait()
            plsc.parallel_loop(0, NG, 1, unroll=4, carry=jnp.int32(0))(
                make_group_body(par))
            pltpu.async_copy(yb.at[par], out_hbm.at[pl.ds(base, CH)],
                             sems_o[par])
        return 0

    lax.fori_loop(0, N_CHUNKS // 2, pair_body, 0)
    for par in range(2):
        base = slab + (N_CHUNKS - 2 + par) * CH
        pltpu.make_async_copy(
            yb.at[par], out_hbm.at[pl.ds(base, CH)], sems_o[par]).wait()


def kernel(x, c, w, m):
    m16 = jnp.broadcast_to(m, (N_DIMS,))
    mesh = plsc.VectorSubcoreMesh(core_axis_name="c", subcore_axis_name="s")
    f = pl.kernel(
        _sc_body,
        out_type=jax.ShapeDtypeStruct((N_ROWS,), jnp.float32),
        mesh=mesh,
        compiler_params=pltpu.CompilerParams(
            needs_layout_passes=False, use_tc_tiling_on_sc=False),
        scratch_types=[
            pltpu.VMEM((2, CH, N_DIMS), jnp.float32),
            pltpu.VMEM((2, CH), jnp.float32),
            pltpu.VMEM((3, N_DIMS), jnp.float32),
            pltpu.VMEM((2, NG, N_DIMS, N_DIMS + 1), jnp.float32),
            pltpu.SemaphoreType.DMA,
            pltpu.SemaphoreType.DMA,
            pltpu.SemaphoreType.DMA,
            pltpu.SemaphoreType.DMA,
        ],
    )
    return f(x, c, w, m16)


# speedup vs baseline: 1.4169x; 1.1017x over previous
import jax
import jax.numpy as jnp
import numpy as np
from jax import lax
from jax.experimental import pallas as pl
from jax.experimental.pallas import tpu as pltpu
from jax.experimental.pallas import tpu_sc as plsc

N_ROWS = 1048576
N_DIMS = 16
NC, NS = 2, 16
NW = NC * NS
ROWS_PER_W = N_ROWS // NW
CH = 512
NG = CH // N_DIMS
N_CHUNKS = ROWS_PER_W // CH
K_SUP = float(np.sqrt(-np.log(0.01)))
K2 = K_SUP * K_SUP
PENALTY = 200.0


def _sc_body(x_hbm, c_hbm, w_hbm, m_hbm, out_hbm,
             xb, yb, pb, sb, si0, si1, so0, so1):
    wid = lax.axis_index("s") * NC + lax.axis_index("c")
    slab = wid * ROWS_PER_W

    pltpu.sync_copy(c_hbm, pb.at[0])
    pltpu.sync_copy(w_hbm, pb.at[1])
    pltpu.sync_copy(m_hbm, pb.at[2])
    cv = pb[0]
    wv = pb[1]
    mv = pb[2]
    iw = 1.0 / wv
    k2v = jnp.where(wv > 0.0, jnp.float32(K2), jnp.float32(-1.0))
    row_iota = lax.iota(jnp.int32, N_DIMS)
    cols = [row_iota * 0 + d for d in range(N_DIMS)]
    sems_i = (si0, si1)
    sems_o = (so0, so1)

    def make_group_body(par):
        parv = row_iota * 0 + par

        def group_body(j, _):
            jbase = j * N_DIMS
            vs = [xb[par, jbase + r] for r in range(N_DIMS)]
            ts = [(v - cv) * iw for v in vs]
            qs = [t * t for t in ts]
            qqs = [jnp.where(q < k2v, q, jnp.float32(PENALTY)) for q in qs]
            for r in range(N_DIMS):
                sb[par, j, r, pl.ds(0, N_DIMS)] = qqs[r]
            jv = row_iota * 0 + j
            g = [plsc.load_gather(sb, [parv, jv, row_iota, cols[d]])
                 for d in range(N_DIMS)]
            while len(g) > 1:
                g = [g[i] + g[i + 1] for i in range(0, len(g), 2)]
            yb[par, pl.ds(jbase, N_DIMS)] = mv * jnp.exp(-g[0])
            return 0

        return group_body

    pltpu.async_copy(x_hbm.at[pl.ds(slab, CH)], xb.at[0], si0)

    def pair_body(p, _):
        for par in range(2):
            k = 2 * p + par
            base = slab + k * CH
            @pl.when(k + 1 < N_CHUNKS)
            def _():
                pltpu.async_copy(
                    x_hbm.at[pl.ds(base + CH, CH)], xb.at[1 - par],
                    sems_i[1 - par])
            pltpu.make_async_copy(
                x_hbm.at[pl.ds(base, CH)], xb.at[par], sems_i[par]).wait()
            @pl.when(k >= 2)
            def _():
                pltpu.make_async_copy(
                    yb.at[par], out_hbm.at[pl.ds(base, CH)],
                    sems_o[par]).wait()
            plsc.parallel_loop(0, NG, 1, unroll=2, carry=jnp.int32(0))(
                make_group_body(par))
            pltpu.async_copy(yb.at[par], out_hbm.at[pl.ds(base, CH)],
                             sems_o[par])
        return 0

    lax.fori_loop(0, N_CHUNKS // 2, pair_body, 0)
    for par in range(2):
        base = slab + (N_CHUNKS - 2 + par) * CH
        pltpu.make_async_copy(
            yb.at[par], out_hbm.at[pl.ds(base, CH)], sems_o[par]).wait()


def kernel(x, c, w, m):
    m16 = jnp.broadcast_to(m, (N_DIMS,))
    mesh = plsc.VectorSubcoreMesh(core_axis_name="c", subcore_axis_name="s")
    f = pl.kernel(
        _sc_body,
        out_type=jax.ShapeDtypeStruct((N_ROWS,), jnp.float32),
        mesh=mesh,
        compiler_params=pltpu.CompilerParams(
            needs_layout_passes=False, use_tc_tiling_on_sc=False),
        scratch_types=[
            pltpu.VMEM((2, CH, N_DIMS), jnp.float32),
            pltpu.VMEM((2, CH), jnp.float32),
            pltpu.VMEM((3, N_DIMS), jnp.float32),
            pltpu.VMEM((2, NG, N_DIMS, N_DIMS + 1), jnp.float32),
            pltpu.SemaphoreType.DMA,
            pltpu.SemaphoreType.DMA,
            pltpu.SemaphoreType.DMA,
            pltpu.SemaphoreType.DMA,
        ],
    )
    return f(x, c, w, m16)
